# baseline (speedup 1.0000x reference)
import functools

import jax
import jax.numpy as jnp
from jax.experimental import pallas as pl
from jax.experimental.pallas import tpu as pltpu


def _rup(x, m):
    return -(-x // m) * m


def _fused_body(n_l, bt, lt, k1, k2, lc1,
                x_ref, w1_ref, b1_ref, w2_ref, b2_ref, wd_ref, bd_ref,
                w3_ref, b3_ref, wx_ref, bx_ref, wa_ref, ba_ref,
                o_ref, acc_ref, h2_ref):
    l = pl.program_id(1)
    cout2 = h2_ref.shape[1]

    @pl.when(l == 0)
    def _init():
        acc_ref[...] = jnp.zeros_like(acc_ref)

    w1 = w1_ref[...]
    w2 = w2_ref[...]
    b1 = b1_ref[...]
    b2 = b2_ref[...]

    def conv_sample(s, carry):
        xs = x_ref[s]
        im1 = jnp.concatenate([xs[:, k:k + lc1] for k in range(k1)], axis=0)
        h1 = jnp.dot(w1, im1, preferred_element_type=jnp.float32) + b1
        h1 = jnp.maximum(h1, 0.0)
        p = jnp.dot(w2, h1, preferred_element_type=jnp.float32)
        h2 = b2 + p[0:cout2, 0:lt]
        for k in range(1, k2):
            h2 = h2 + p[k * cout2:(k + 1) * cout2, k:k + lt]
        h2_ref[s] = jnp.maximum(h2, 0.0)
        return carry

    jax.lax.fori_loop(0, bt, conv_sample, 0, unroll=4)

    part = jnp.zeros(acc_ref.shape, jnp.float32)
    for c in range(cout2):
        part = part + jnp.dot(h2_ref[:, c, :],
                              wd_ref[c].astype(jnp.float32),
                              preferred_element_type=jnp.float32)
    acc_ref[...] += part

    @pl.when(l == n_l - 1)
    def _tail():
        h = jnp.maximum(acc_ref[...] + bd_ref[...], 0.0)
        h = jnp.maximum(
            jnp.dot(h, w3_ref[...], preferred_element_type=jnp.float32)
            + b3_ref[...], 0.0)
        h = jnp.maximum(
            jnp.dot(h, wx_ref[...], preferred_element_type=jnp.float32)
            + bx_ref[...], 0.0)
        z = jnp.dot(h, wa_ref[...],
                    preferred_element_type=jnp.float32) + ba_ref[...]
        o_ref[...] = jax.nn.sigmoid(z)


def kernel(x, conv1_w, conv1_b, conv2_w, conv2_b, liner1_w, liner1_b,
           liner3_w, liner3_b, lin_only_x1_w, lin_only_x1_b,
           linear_a2_w, linear_a2_b):
    B, Cin, Lin = x.shape
    Cout1, _, K1 = conv1_w.shape
    Cout2, _, K2 = conv2_w.shape
    L1 = Lin - K1 + 1
    L2 = L1 - K2 + 1
    H1 = liner1_w.shape[0]

    LT = 1024
    BT = 32 if B % 32 == 0 else B
    L2p = _rup(L2, LT)
    n_l = L2p // LT
    lc1 = LT + K2 - 1
    lcin = lc1 + K1 - 1
    Lin_p = L2p + K1 + K2 - 2

    xp = jnp.pad(x.astype(jnp.float32), ((0, 0), (0, 0), (0, Lin_p - Lin)))
    xch = jnp.stack([xp[:, :, l * LT:l * LT + lcin] for l in range(n_l)],
                    axis=1)

    w1m = jnp.transpose(conv1_w, (0, 2, 1)).reshape(Cout1, K1 * Cin)
    w1m = w1m.astype(jnp.float32)
    b1 = conv1_b.reshape(Cout1, 1).astype(jnp.float32)
    w2s = jnp.transpose(conv2_w, (2, 0, 1)).reshape(K2 * Cout2, Cout1)
    w2s = w2s.astype(jnp.float32)
    b2 = conv2_b.reshape(Cout2, 1).astype(jnp.float32)

    wd = liner1_w.T.reshape(Cout2, L2, H1)
    wd = jnp.pad(wd, ((0, 0), (0, L2p - L2), (0, 0))).astype(jnp.bfloat16)
    bd = liner1_b.reshape(1, H1).astype(jnp.float32)

    w3 = liner3_w.T.astype(jnp.float32)
    b3 = liner3_b[None, :].astype(jnp.float32)
    wx = lin_only_x1_w.T.astype(jnp.float32)
    bx = lin_only_x1_b[None, :].astype(jnp.float32)
    wa = linear_a2_w.T.astype(jnp.float32)
    ba = linear_a2_b[None, :].astype(jnp.float32)

    body = functools.partial(_fused_body, n_l, BT, LT, K1, K2, lc1)

    out = pl.pallas_call(
        body,
        out_shape=jax.ShapeDtypeStruct((B, 1), jnp.float32),
        grid=(B // BT, n_l),
        in_specs=[
            pl.BlockSpec((BT, None, Cin, lcin), lambda i, l: (i, l, 0, 0)),
            pl.BlockSpec(w1m.shape, lambda i, l: (0, 0)),
            pl.BlockSpec(b1.shape, lambda i, l: (0, 0)),
            pl.BlockSpec(w2s.shape, lambda i, l: (0, 0)),
            pl.BlockSpec(b2.shape, lambda i, l: (0, 0)),
            pl.BlockSpec((Cout2, LT, H1), lambda i, l: (0, l, 0)),
            pl.BlockSpec(bd.shape, lambda i, l: (0, 0)),
            pl.BlockSpec(w3.shape, lambda i, l: (0, 0)),
            pl.BlockSpec(b3.shape, lambda i, l: (0, 0)),
            pl.BlockSpec(wx.shape, lambda i, l: (0, 0)),
            pl.BlockSpec(bx.shape, lambda i, l: (0, 0)),
            pl.BlockSpec(wa.shape, lambda i, l: (0, 0)),
            pl.BlockSpec(ba.shape, lambda i, l: (0, 0)),
        ],
        out_specs=pl.BlockSpec((BT, 1), lambda i, l: (i, 0)),
        scratch_shapes=[
            pltpu.VMEM((BT, H1), jnp.float32),
            pltpu.VMEM((BT, Cout2, LT), jnp.float32),
        ],
        compiler_params=pltpu.CompilerParams(
            dimension_semantics=("parallel", "arbitrary"),
            vmem_limit_bytes=100 * 1024 * 1024),
    )(xch, w1m, b1, w2s, b2, wd, bd, w3, b3, wx, bx, wa, ba)
    return out


# speedup vs baseline: 1.0521x; 1.0521x over previous
import functools

import jax
import jax.numpy as jnp
from jax.experimental import pallas as pl
from jax.experimental.pallas import tpu as pltpu


def _rup(x, m):
    return -(-x // m) * m


def _fused_body(n_l, bt, lt, k2, lc1,
                x_ref, w1_ref, b1_ref, w2_ref, b2_ref, wd_ref, bd_ref,
                w3_ref, b3_ref, wx_ref, bx_ref, wa_ref, ba_ref,
                o_ref, acc_ref, h2_ref):
    l = pl.program_id(1)
    cout2 = h2_ref.shape[1]

    @pl.when(l == 0)
    def _init():
        acc_ref[...] = jnp.zeros_like(acc_ref)

    w1 = w1_ref[...]
    w2 = w2_ref[...]
    b1 = b1_ref[...]
    b2 = b2_ref[...]

    def conv_sample(s, carry):
        im1 = x_ref[s]
        h1 = jnp.dot(w1, im1, preferred_element_type=jnp.float32) + b1
        h1 = jnp.maximum(h1, 0.0)
        p = jnp.dot(w2, h1, preferred_element_type=jnp.float32)
        h2 = b2 + p[0:cout2, 0:lt]
        for k in range(1, k2):
            h2 = h2 + p[k * cout2:(k + 1) * cout2, k:k + lt]
        h2_ref[s] = jnp.maximum(h2, 0.0)
        return carry

    jax.lax.fori_loop(0, bt, conv_sample, 0, unroll=4)

    part = jnp.zeros(acc_ref.shape, jnp.float32)
    for c in range(cout2):
        part = part + jnp.dot(h2_ref[:, c, :],
                              wd_ref[c].astype(jnp.float32),
                              preferred_element_type=jnp.float32)
    acc_ref[...] += part

    @pl.when(l == n_l - 1)
    def _tail():
        h = jnp.maximum(acc_ref[...] + bd_ref[...], 0.0)
        h = jnp.maximum(
            jnp.dot(h, w3_ref[...], preferred_element_type=jnp.float32)
            + b3_ref[...], 0.0)
        h = jnp.maximum(
            jnp.dot(h, wx_ref[...], preferred_element_type=jnp.float32)
            + bx_ref[...], 0.0)
        z = jnp.dot(h, wa_ref[...],
                    preferred_element_type=jnp.float32) + ba_ref[...]
        o_ref[...] = jax.nn.sigmoid(z)


def kernel(x, conv1_w, conv1_b, conv2_w, conv2_b, liner1_w, liner1_b,
           liner3_w, liner3_b, lin_only_x1_w, lin_only_x1_b,
           linear_a2_w, linear_a2_b):
    B, Cin, Lin = x.shape
    Cout1, _, K1 = conv1_w.shape
    Cout2, _, K2 = conv2_w.shape
    L1 = Lin - K1 + 1
    L2 = L1 - K2 + 1
    H1 = liner1_w.shape[0]

    LT = 1024
    BT = 32 if B % 32 == 0 else B
    L2p = _rup(L2, LT)
    n_l = L2p // LT
    lc1 = LT + K2 - 1
    lcin = lc1 + K1 - 1
    Lin_p = L2p + K1 + K2 - 2

    xb = jnp.pad(x.astype(jnp.float32),
                 ((0, 0), (0, 0), (0, Lin_p - Lin))).astype(jnp.bfloat16)
    chunks = []
    for l in range(n_l):
        win = xb[:, :, l * LT:l * LT + lcin]
        chunks.append(jnp.stack([win[:, :, k:k + lc1] for k in range(K1)],
                                axis=1).reshape(B, K1 * Cin, lc1))
    xim = jnp.stack(chunks, axis=1)

    w1m = jnp.transpose(conv1_w, (0, 2, 1)).reshape(Cout1, K1 * Cin)
    w1m = w1m.astype(jnp.bfloat16)
    b1 = conv1_b.reshape(Cout1, 1).astype(jnp.float32)
    w2s = jnp.transpose(conv2_w, (2, 0, 1)).reshape(K2 * Cout2, Cout1)
    w2s = w2s.astype(jnp.float32)
    b2 = conv2_b.reshape(Cout2, 1).astype(jnp.float32)

    wd = liner1_w.T.reshape(Cout2, L2, H1)
    wd = jnp.pad(wd, ((0, 0), (0, L2p - L2), (0, 0))).astype(jnp.bfloat16)
    bd = liner1_b.reshape(1, H1).astype(jnp.float32)

    w3 = liner3_w.T.astype(jnp.float32)
    b3 = liner3_b[None, :].astype(jnp.float32)
    wx = lin_only_x1_w.T.astype(jnp.float32)
    bx = lin_only_x1_b[None, :].astype(jnp.float32)
    wa = linear_a2_w.T.astype(jnp.float32)
    ba = linear_a2_b[None, :].astype(jnp.float32)

    body = functools.partial(_fused_body, n_l, BT, LT, K2, lc1)

    out = pl.pallas_call(
        body,
        out_shape=jax.ShapeDtypeStruct((B, 1), jnp.float32),
        grid=(B // BT, n_l),
        in_specs=[
            pl.BlockSpec((BT, None, K1 * Cin, lc1), lambda i, l: (i, l, 0, 0)),
            pl.BlockSpec(w1m.shape, lambda i, l: (0, 0)),
            pl.BlockSpec(b1.shape, lambda i, l: (0, 0)),
            pl.BlockSpec(w2s.shape, lambda i, l: (0, 0)),
            pl.BlockSpec(b2.shape, lambda i, l: (0, 0)),
            pl.BlockSpec((Cout2, LT, H1), lambda i, l: (0, l, 0)),
            pl.BlockSpec(bd.shape, lambda i, l: (0, 0)),
            pl.BlockSpec(w3.shape, lambda i, l: (0, 0)),
            pl.BlockSpec(b3.shape, lambda i, l: (0, 0)),
            pl.BlockSpec(wx.shape, lambda i, l: (0, 0)),
            pl.BlockSpec(bx.shape, lambda i, l: (0, 0)),
            pl.BlockSpec(wa.shape, lambda i, l: (0, 0)),
            pl.BlockSpec(ba.shape, lambda i, l: (0, 0)),
        ],
        out_specs=pl.BlockSpec((BT, 1), lambda i, l: (i, 0)),
        scratch_shapes=[
            pltpu.VMEM((BT, H1), jnp.float32),
            pltpu.VMEM((BT, Cout2, LT), jnp.float32),
        ],
        compiler_params=pltpu.CompilerParams(
            dimension_semantics=("parallel", "arbitrary"),
            vmem_limit_bytes=100 * 1024 * 1024),
    )(xim, w1m, b1, w2s, b2, wd, bd, w3, b3, wx, bx, wa, ba)
    return out


# speedup vs baseline: 1.0766x; 1.0232x over previous
import functools

import jax
import jax.numpy as jnp
from jax.experimental import pallas as pl
from jax.experimental.pallas import tpu as pltpu


def _rup(x, m):
    return -(-x // m) * m


def _fused_body(n_l, bt, lt, k2, lc1,
                x_ref, w1_ref, b1_ref, w2_ref, b2_ref, wd_ref, bd_ref,
                w3_ref, b3_ref, wx_ref, bx_ref, wa_ref, ba_ref,
                o_ref, acc_ref, h2_ref):
    l = pl.program_id(1)
    cout2 = h2_ref.shape[1]

    @pl.when(l == 0)
    def _init():
        acc_ref[...] = jnp.zeros_like(acc_ref)

    w1 = w1_ref[...]
    w2 = w2_ref[...]
    b1 = b1_ref[...]
    b2 = b2_ref[...]

    def conv_sample(s, carry):
        im1 = x_ref[s]
        h1 = jnp.dot(w1, im1, preferred_element_type=jnp.float32) + b1
        h1 = jnp.maximum(h1, 0.0)
        p = jnp.dot(w2, h1, preferred_element_type=jnp.float32)
        h2 = b2 + p[0:cout2, 0:lt]
        for k in range(1, k2):
            h2 = h2 + p[k * cout2:(k + 1) * cout2, k:k + lt]
        h2_ref[s] = jnp.maximum(h2, 0.0)
        return carry

    jax.lax.fori_loop(0, bt, conv_sample, 0, unroll=8)

    part = jnp.zeros(acc_ref.shape, jnp.float32)
    for c in range(cout2):
        part = part + jnp.dot(h2_ref[:, c, :],
                              wd_ref[c].astype(jnp.float32),
                              preferred_element_type=jnp.float32)
    acc_ref[...] += part

    @pl.when(l == n_l - 1)
    def _tail():
        h = jnp.maximum(acc_ref[...] + bd_ref[...], 0.0)
        h = jnp.maximum(
            jnp.dot(h, w3_ref[...], preferred_element_type=jnp.float32)
            + b3_ref[...], 0.0)
        h = jnp.maximum(
            jnp.dot(h, wx_ref[...], preferred_element_type=jnp.float32)
            + bx_ref[...], 0.0)
        z = jnp.dot(h, wa_ref[...],
                    preferred_element_type=jnp.float32) + ba_ref[...]
        o_ref[...] = jax.nn.sigmoid(z)


def kernel(x, conv1_w, conv1_b, conv2_w, conv2_b, liner1_w, liner1_b,
           liner3_w, liner3_b, lin_only_x1_w, lin_only_x1_b,
           linear_a2_w, linear_a2_b):
    B, Cin, Lin = x.shape
    Cout1, _, K1 = conv1_w.shape
    Cout2, _, K2 = conv2_w.shape
    L1 = Lin - K1 + 1
    L2 = L1 - K2 + 1
    H1 = liner1_w.shape[0]

    LT = 1024
    BT = 64 if B % 64 == 0 else B
    L2p = _rup(L2, LT)
    n_l = L2p // LT
    lc1 = LT + K2 - 1
    lcin = lc1 + K1 - 1
    Lin_p = L2p + K1 + K2 - 2

    xb = jnp.pad(x.astype(jnp.float32),
                 ((0, 0), (0, 0), (0, Lin_p - Lin))).astype(jnp.bfloat16)
    chunks = []
    for l in range(n_l):
        win = xb[:, :, l * LT:l * LT + lcin]
        chunks.append(jnp.stack([win[:, :, k:k + lc1] for k in range(K1)],
                                axis=1).reshape(B, K1 * Cin, lc1))
    xim = jnp.stack(chunks, axis=1)

    w1m = jnp.transpose(conv1_w, (0, 2, 1)).reshape(Cout1, K1 * Cin)
    w1m = w1m.astype(jnp.bfloat16)
    b1 = conv1_b.reshape(Cout1, 1).astype(jnp.float32)
    w2s = jnp.transpose(conv2_w, (2, 0, 1)).reshape(K2 * Cout2, Cout1)
    w2s = w2s.astype(jnp.float32)
    b2 = conv2_b.reshape(Cout2, 1).astype(jnp.float32)

    wd = liner1_w.T.reshape(Cout2, L2, H1)
    wd = jnp.pad(wd, ((0, 0), (0, L2p - L2), (0, 0))).astype(jnp.bfloat16)
    bd = liner1_b.reshape(1, H1).astype(jnp.float32)

    w3 = liner3_w.T.astype(jnp.float32)
    b3 = liner3_b[None, :].astype(jnp.float32)
    wx = lin_only_x1_w.T.astype(jnp.float32)
    bx = lin_only_x1_b[None, :].astype(jnp.float32)
    wa = linear_a2_w.T.astype(jnp.float32)
    ba = linear_a2_b[None, :].astype(jnp.float32)

    body = functools.partial(_fused_body, n_l, BT, LT, K2, lc1)

    out = pl.pallas_call(
        body,
        out_shape=jax.ShapeDtypeStruct((B, 1), jnp.float32),
        grid=(B // BT, n_l),
        in_specs=[
            pl.BlockSpec((BT, None, K1 * Cin, lc1), lambda i, l: (i, l, 0, 0)),
            pl.BlockSpec(w1m.shape, lambda i, l: (0, 0)),
            pl.BlockSpec(b1.shape, lambda i, l: (0, 0)),
            pl.BlockSpec(w2s.shape, lambda i, l: (0, 0)),
            pl.BlockSpec(b2.shape, lambda i, l: (0, 0)),
            pl.BlockSpec((Cout2, LT, H1), lambda i, l: (0, l, 0)),
            pl.BlockSpec(bd.shape, lambda i, l: (0, 0)),
            pl.BlockSpec(w3.shape, lambda i, l: (0, 0)),
            pl.BlockSpec(b3.shape, lambda i, l: (0, 0)),
            pl.BlockSpec(wx.shape, lambda i, l: (0, 0)),
            pl.BlockSpec(bx.shape, lambda i, l: (0, 0)),
            pl.BlockSpec(wa.shape, lambda i, l: (0, 0)),
            pl.BlockSpec(ba.shape, lambda i, l: (0, 0)),
        ],
        out_specs=pl.BlockSpec((BT, 1), lambda i, l: (i, 0)),
        scratch_shapes=[
            pltpu.VMEM((BT, H1), jnp.float32),
            pltpu.VMEM((BT, Cout2, LT), jnp.float32),
        ],
        compiler_params=pltpu.CompilerParams(
            dimension_semantics=("parallel", "arbitrary"),
            vmem_limit_bytes=100 * 1024 * 1024),
    )(xim, w1m, b1, w2s, b2, wd, bd, w3, b3, wx, bx, wa, ba)
    return out


# speedup vs baseline: 1.0996x; 1.0214x over previous
import functools

import jax
import jax.numpy as jnp
from jax.experimental import pallas as pl
from jax.experimental.pallas import tpu as pltpu


def _rup(x, m):
    return -(-x // m) * m


def _fused_body(n_l, bt, lt, k2, lc1,
                x_ref, w1_ref, b1_ref, w2_ref, b2_ref, wd_ref, bd_ref,
                w3_ref, b3_ref, wx_ref, bx_ref, wa_ref, ba_ref,
                o_ref, acc_ref, h2_ref):
    l = pl.program_id(1)
    cout2 = h2_ref.shape[1]

    @pl.when(l == 0)
    def _init():
        acc_ref[...] = jnp.zeros_like(acc_ref)

    w1 = w1_ref[...]
    w2 = w2_ref[...]
    b1 = b1_ref[...]
    b2 = b2_ref[...]

    def conv_sample(s, carry):
        im1 = x_ref[s]
        h1 = jnp.dot(w1, im1, preferred_element_type=jnp.float32) + b1
        h1 = jnp.maximum(h1, 0.0)
        p = jnp.dot(w2, h1, preferred_element_type=jnp.float32)
        h2 = b2 + p[0:cout2, 0:lt]
        for k in range(1, k2):
            h2 = h2 + p[k * cout2:(k + 1) * cout2, k:k + lt]
        h2_ref[s] = jnp.maximum(h2, 0.0)
        return carry

    jax.lax.fori_loop(0, bt, conv_sample, 0, unroll=8)

    part = jnp.zeros(acc_ref.shape, jnp.float32)
    for c in range(cout2):
        part = part + jnp.dot(h2_ref[:, c, :],
                              wd_ref[c].astype(jnp.float32),
                              preferred_element_type=jnp.float32)
    acc_ref[...] += part

    @pl.when(l == n_l - 1)
    def _tail():
        h = jnp.maximum(acc_ref[...] + bd_ref[...], 0.0)
        h = jnp.maximum(
            jnp.dot(h, w3_ref[...], preferred_element_type=jnp.float32)
            + b3_ref[...], 0.0)
        h = jnp.maximum(
            jnp.dot(h, wx_ref[...], preferred_element_type=jnp.float32)
            + bx_ref[...], 0.0)
        z = jnp.dot(h, wa_ref[...],
                    preferred_element_type=jnp.float32) + ba_ref[...]
        o_ref[...] = jax.nn.sigmoid(z)


def kernel(x, conv1_w, conv1_b, conv2_w, conv2_b, liner1_w, liner1_b,
           liner3_w, liner3_b, lin_only_x1_w, lin_only_x1_b,
           linear_a2_w, linear_a2_b):
    B, Cin, Lin = x.shape
    Cout1, _, K1 = conv1_w.shape
    Cout2, _, K2 = conv2_w.shape
    L1 = Lin - K1 + 1
    L2 = L1 - K2 + 1
    H1 = liner1_w.shape[0]

    LT = 1536
    BT = 64 if B % 64 == 0 else B
    L2p = _rup(L2, LT)
    n_l = L2p // LT
    lc1 = LT + K2 - 1
    lcin = lc1 + K1 - 1
    Lin_p = L2p + K1 + K2 - 2

    xb = jnp.pad(x.astype(jnp.float32),
                 ((0, 0), (0, 0), (0, Lin_p - Lin))).astype(jnp.bfloat16)
    chunks = []
    for l in range(n_l):
        win = xb[:, :, l * LT:l * LT + lcin]
        chunks.append(jnp.stack([win[:, :, k:k + lc1] for k in range(K1)],
                                axis=1).reshape(B, K1 * Cin, lc1))
    xim = jnp.stack(chunks, axis=1)

    w1m = jnp.transpose(conv1_w, (0, 2, 1)).reshape(Cout1, K1 * Cin)
    w1m = w1m.astype(jnp.bfloat16)
    b1 = conv1_b.reshape(Cout1, 1).astype(jnp.float32)
    w2s = jnp.transpose(conv2_w, (2, 0, 1)).reshape(K2 * Cout2, Cout1)
    w2s = w2s.astype(jnp.float32)
    b2 = conv2_b.reshape(Cout2, 1).astype(jnp.float32)

    wd = liner1_w.T.reshape(Cout2, L2, H1)
    wd = jnp.pad(wd, ((0, 0), (0, L2p - L2), (0, 0))).astype(jnp.bfloat16)
    bd = liner1_b.reshape(1, H1).astype(jnp.float32)

    w3 = liner3_w.T.astype(jnp.float32)
    b3 = liner3_b[None, :].astype(jnp.float32)
    wx = lin_only_x1_w.T.astype(jnp.float32)
    bx = lin_only_x1_b[None, :].astype(jnp.float32)
    wa = linear_a2_w.T.astype(jnp.float32)
    ba = linear_a2_b[None, :].astype(jnp.float32)

    body = functools.partial(_fused_body, n_l, BT, LT, K2, lc1)

    out = pl.pallas_call(
        body,
        out_shape=jax.ShapeDtypeStruct((B, 1), jnp.float32),
        grid=(B // BT, n_l),
        in_specs=[
            pl.BlockSpec((BT, None, K1 * Cin, lc1), lambda i, l: (i, l, 0, 0)),
            pl.BlockSpec(w1m.shape, lambda i, l: (0, 0)),
            pl.BlockSpec(b1.shape, lambda i, l: (0, 0)),
            pl.BlockSpec(w2s.shape, lambda i, l: (0, 0)),
            pl.BlockSpec(b2.shape, lambda i, l: (0, 0)),
            pl.BlockSpec((Cout2, LT, H1), lambda i, l: (0, l, 0)),
            pl.BlockSpec(bd.shape, lambda i, l: (0, 0)),
            pl.BlockSpec(w3.shape, lambda i, l: (0, 0)),
            pl.BlockSpec(b3.shape, lambda i, l: (0, 0)),
            pl.BlockSpec(wx.shape, lambda i, l: (0, 0)),
            pl.BlockSpec(bx.shape, lambda i, l: (0, 0)),
            pl.BlockSpec(wa.shape, lambda i, l: (0, 0)),
            pl.BlockSpec(ba.shape, lambda i, l: (0, 0)),
        ],
        out_specs=pl.BlockSpec((BT, 1), lambda i, l: (i, 0)),
        scratch_shapes=[
            pltpu.VMEM((BT, H1), jnp.float32),
            pltpu.VMEM((BT, Cout2, LT), jnp.float32),
        ],
        compiler_params=pltpu.CompilerParams(
            dimension_semantics=("parallel", "arbitrary"),
            vmem_limit_bytes=100 * 1024 * 1024),
    )(xim, w1m, b1, w2s, b2, wd, bd, w3, b3, wx, bx, wa, ba)
    return out
